# Initial kernel scaffold; baseline (speedup 1.0000x reference)
import jax
import jax.numpy as jnp
from jax.experimental import pallas as pl


def kernel(x, conv_w, gamma, beta):
    raise NotImplementedError("write your pallas kernel here")


# speedup vs baseline: 1.0726x; 1.0726x over previous
import functools

import jax
import jax.numpy as jnp
from jax.experimental import pallas as pl
from jax.experimental.pallas import tpu as pltpu

BN_EPS = 1e-5


def _gram_kernel(xs_ref, g_ref, s_ref):
    i = pl.program_id(0)

    @pl.when(i == 0)
    def _():
        g_ref[...] = jnp.zeros_like(g_ref)
        s_ref[...] = jnp.zeros_like(s_ref)

    for b in range(xs_ref.shape[0]):
        xb = xs_ref[b]
        g_ref[...] += jax.lax.dot_general(
            xb, xb, (((1,), (1,)), ((), ())),
            preferred_element_type=jnp.float32)
        s_ref[...] += jnp.sum(xb.astype(jnp.float32), axis=1, keepdims=True)


def _conv_bn_kernel(xs_ref, w_ref, scale_ref, shift_ref, o_ref):
    w = w_ref[...]
    scale = scale_ref[...]
    shift = shift_ref[...]
    for b in range(xs_ref.shape[0]):
        y = jnp.dot(w, xs_ref[b], preferred_element_type=jnp.float32)
        o_ref[b] = y * scale + shift


@jax.jit
def _forward(x, conv_w, gamma, beta):
    n, c_in, h, w = x.shape
    c_out = conv_w.shape[0]

    xs = x[:, :, ::2, ::2]
    ho, wo = xs.shape[2], xs.shape[3]
    p = ho * wo
    xs = xs.reshape(n, c_in, p).astype(jnp.bfloat16)
    w_mat = conv_w.reshape(c_out, c_in).astype(jnp.bfloat16)

    tile_n = 4
    while n % tile_n != 0 and tile_n > 1:
        tile_n //= 2
    steps = n // tile_n

    gm, sx = pl.pallas_call(
        _gram_kernel,
        grid=(steps,),
        in_specs=[
            pl.BlockSpec((tile_n, c_in, p), lambda i: (i, 0, 0)),
        ],
        out_specs=[
            pl.BlockSpec((c_in, c_in), lambda i: (0, 0)),
            pl.BlockSpec((c_in, 1), lambda i: (0, 0)),
        ],
        out_shape=[
            jax.ShapeDtypeStruct((c_in, c_in), jnp.float32),
            jax.ShapeDtypeStruct((c_in, 1), jnp.float32),
        ],
        compiler_params=pltpu.CompilerParams(
            dimension_semantics=(pltpu.ARBITRARY,),
        ),
        cost_estimate=pl.CostEstimate(
            flops=2 * n * p * c_in * c_in,
            transcendentals=0,
            bytes_accessed=n * c_in * p * 2 + (c_in * c_in + c_in) * 4,
        ),
    )(xs)

    m = float(n * p)
    wq = w_mat.astype(jnp.float32)
    mean = (wq @ sx) / m
    ey2 = jnp.sum((wq @ gm) * wq, axis=1, keepdims=True) / m
    var = jnp.maximum(ey2 - mean * mean, 0.0)
    inv_std = jax.lax.rsqrt(var + BN_EPS)
    scale = gamma.astype(jnp.float32).reshape(c_out, 1) * inv_std
    shift = beta.astype(jnp.float32).reshape(c_out, 1) - mean * scale

    out_flat = pl.pallas_call(
        _conv_bn_kernel,
        grid=(steps,),
        in_specs=[
            pl.BlockSpec((tile_n, c_in, p), lambda i: (i, 0, 0)),
            pl.BlockSpec((c_out, c_in), lambda i: (0, 0)),
            pl.BlockSpec((c_out, 1), lambda i: (0, 0)),
            pl.BlockSpec((c_out, 1), lambda i: (0, 0)),
        ],
        out_specs=pl.BlockSpec(
            (tile_n, c_out, p), lambda i: (i, 0, 0)),
        out_shape=jax.ShapeDtypeStruct((n, c_out, p), x.dtype),
        compiler_params=pltpu.CompilerParams(
            dimension_semantics=(pltpu.PARALLEL,),
        ),
        cost_estimate=pl.CostEstimate(
            flops=2 * n * p * c_in * c_out + 2 * n * p * c_out,
            transcendentals=0,
            bytes_accessed=n * c_in * p * 2 + c_out * c_in * 2
            + n * c_out * p * 4,
        ),
    )(xs, w_mat, scale, shift)

    return out_flat.reshape(n, c_out, ho, wo)


def kernel(x, conv_w, gamma, beta):
    return _forward(x, conv_w, gamma, beta)


# speedup vs baseline: 1.5263x; 1.4230x over previous
import functools

import jax
import jax.numpy as jnp
from jax.experimental import pallas as pl
from jax.experimental.pallas import tpu as pltpu

BN_EPS = 1e-5


def _subsample_kernel(x_ref, sel_ref, o_ref, scratch_ref):
    _, c, hh, ww = x_ref.shape
    wo = ww // 2
    xb = x_ref[0].astype(jnp.bfloat16)
    xm = xb.reshape(c * hh, ww)
    y = jnp.dot(xm, sel_ref[...], preferred_element_type=jnp.float32)
    scratch_ref[...] = y.reshape(c, hh, wo)
    even = scratch_ref[:, pl.ds(0, hh // 2, 2), :]
    o_ref[0] = even.astype(o_ref.dtype)


def _gram_kernel(xs_ref, g_ref, s_ref):
    i = pl.program_id(0)

    @pl.when(i == 0)
    def _():
        g_ref[...] = jnp.zeros_like(g_ref)
        s_ref[...] = jnp.zeros_like(s_ref)

    for b in range(xs_ref.shape[0]):
        xb = xs_ref[b]
        g_ref[...] += jax.lax.dot_general(
            xb, xb, (((1,), (1,)), ((), ())),
            preferred_element_type=jnp.float32)
        s_ref[...] += jnp.sum(xb.astype(jnp.float32), axis=1, keepdims=True)


def _conv_bn_kernel(xs_ref, w_ref, scale_ref, shift_ref, o_ref):
    w = w_ref[...]
    scale = scale_ref[...]
    shift = shift_ref[...]
    for b in range(xs_ref.shape[0]):
        y = jnp.dot(w, xs_ref[b], preferred_element_type=jnp.float32)
        o_ref[b] = y * scale + shift


@jax.jit
def _forward(x, conv_w, gamma, beta):
    n, c_in, h, w = x.shape
    c_out = conv_w.shape[0]

    ho, wo = h // 2, w // 2
    p = ho * wo
    w_mat = conv_w.reshape(c_out, c_in).astype(jnp.bfloat16)

    sel = (jnp.arange(w)[:, None] == 2 * jnp.arange(wo)[None, :]).astype(
        jnp.bfloat16)
    xs4 = pl.pallas_call(
        _subsample_kernel,
        grid=(n,),
        in_specs=[
            pl.BlockSpec((1, c_in, h, w), lambda i: (i, 0, 0, 0)),
            pl.BlockSpec((w, wo), lambda i: (0, 0)),
        ],
        out_specs=pl.BlockSpec((1, c_in, ho, wo), lambda i: (i, 0, 0, 0)),
        out_shape=jax.ShapeDtypeStruct((n, c_in, ho, wo), jnp.bfloat16),
        scratch_shapes=[pltpu.VMEM((c_in, h, wo), jnp.float32)],
        compiler_params=pltpu.CompilerParams(
            dimension_semantics=(pltpu.PARALLEL,),
            vmem_limit_bytes=56 * 1024 * 1024,
        ),
        cost_estimate=pl.CostEstimate(
            flops=2 * n * c_in * h * w * wo,
            transcendentals=0,
            bytes_accessed=n * c_in * (h * w * 4 + p * 2),
        ),
    )(x, sel)
    xs = xs4.reshape(n, c_in, p)

    tile_n = 4
    while n % tile_n != 0 and tile_n > 1:
        tile_n //= 2
    steps = n // tile_n

    gm, sx = pl.pallas_call(
        _gram_kernel,
        grid=(steps,),
        in_specs=[
            pl.BlockSpec((tile_n, c_in, p), lambda i: (i, 0, 0)),
        ],
        out_specs=[
            pl.BlockSpec((c_in, c_in), lambda i: (0, 0)),
            pl.BlockSpec((c_in, 1), lambda i: (0, 0)),
        ],
        out_shape=[
            jax.ShapeDtypeStruct((c_in, c_in), jnp.float32),
            jax.ShapeDtypeStruct((c_in, 1), jnp.float32),
        ],
        compiler_params=pltpu.CompilerParams(
            dimension_semantics=(pltpu.ARBITRARY,),
        ),
        cost_estimate=pl.CostEstimate(
            flops=2 * n * p * c_in * c_in,
            transcendentals=0,
            bytes_accessed=n * c_in * p * 2 + (c_in * c_in + c_in) * 4,
        ),
    )(xs)

    m = float(n * p)
    wq = w_mat.astype(jnp.float32)
    mean = (wq @ sx) / m
    ey2 = jnp.sum((wq @ gm) * wq, axis=1, keepdims=True) / m
    var = jnp.maximum(ey2 - mean * mean, 0.0)
    inv_std = jax.lax.rsqrt(var + BN_EPS)
    scale = gamma.astype(jnp.float32).reshape(c_out, 1) * inv_std
    shift = beta.astype(jnp.float32).reshape(c_out, 1) - mean * scale

    out_flat = pl.pallas_call(
        _conv_bn_kernel,
        grid=(steps,),
        in_specs=[
            pl.BlockSpec((tile_n, c_in, p), lambda i: (i, 0, 0)),
            pl.BlockSpec((c_out, c_in), lambda i: (0, 0)),
            pl.BlockSpec((c_out, 1), lambda i: (0, 0)),
            pl.BlockSpec((c_out, 1), lambda i: (0, 0)),
        ],
        out_specs=pl.BlockSpec(
            (tile_n, c_out, p), lambda i: (i, 0, 0)),
        out_shape=jax.ShapeDtypeStruct((n, c_out, p), x.dtype),
        compiler_params=pltpu.CompilerParams(
            dimension_semantics=(pltpu.PARALLEL,),
        ),
        cost_estimate=pl.CostEstimate(
            flops=2 * n * p * c_in * c_out + 2 * n * p * c_out,
            transcendentals=0,
            bytes_accessed=n * c_in * p * 2 + c_out * c_in * 2
            + n * c_out * p * 4,
        ),
    )(xs, w_mat, scale, shift)

    return out_flat.reshape(n, c_out, ho, wo)


def kernel(x, conv_w, gamma, beta):
    return _forward(x, conv_w, gamma, beta)


# speedup vs baseline: 2.9454x; 1.9297x over previous
import functools

import jax
import jax.numpy as jnp
from jax.experimental import pallas as pl
from jax.experimental.pallas import tpu as pltpu

BN_EPS = 1e-5


def _sub_stats_kernel(x_ref, xs_ref, g_ref, s_ref, *, ho, wo, w_full):
    i = pl.program_id(0)

    @pl.when(i == 0)
    def _():
        g_ref[...] = jnp.zeros_like(g_ref)
        s_ref[...] = jnp.zeros_like(s_ref)

    c_in = xs_ref.shape[2]
    s_acc = jnp.zeros((1, c_in), jnp.float32)
    for r in range(ho):
        ev = x_ref[0, pl.ds(w_full * r, wo), 0:c_in]
        xs_ref[0, pl.ds(r * wo, wo), :] = ev.astype(xs_ref.dtype)
        s_acc = s_acc + jnp.sum(ev, axis=0, keepdims=True)
    s_ref[...] += s_acc

    xb = xs_ref[0]
    g_ref[...] += jax.lax.dot_general(
        xb, xb, (((0,), (0,)), ((), ())),
        preferred_element_type=jnp.float32)


def _conv_bn_kernel(xs_ref, w_ref, scale_ref, shift_ref, o_ref):
    w = w_ref[...]
    scale = scale_ref[...]
    shift = shift_ref[...]
    for b in range(xs_ref.shape[0]):
        y = jax.lax.dot_general(
            xs_ref[b], w, (((1,), (1,)), ((), ())),
            preferred_element_type=jnp.float32)
        o_ref[b] = y * scale + shift


@jax.jit
def _forward(x, conv_w, gamma, beta):
    n, c_in, h, w = x.shape
    c_out = conv_w.shape[0]
    ho, wo = h // 2, w // 2
    p = ho * wo

    xt = x.transpose(0, 2, 3, 1).reshape(n, h * w // 2, 2 * c_in)
    w_mat = conv_w.reshape(c_out, c_in).astype(jnp.bfloat16)

    xs, gm, sx = pl.pallas_call(
        functools.partial(_sub_stats_kernel, ho=ho, wo=wo, w_full=w),
        grid=(n,),
        in_specs=[
            pl.BlockSpec((1, h * w // 2, 2 * c_in), lambda i: (i, 0, 0)),
        ],
        out_specs=[
            pl.BlockSpec((1, p, c_in), lambda i: (i, 0, 0)),
            pl.BlockSpec((c_in, c_in), lambda i: (0, 0)),
            pl.BlockSpec((1, c_in), lambda i: (0, 0)),
        ],
        out_shape=[
            jax.ShapeDtypeStruct((n, p, c_in), jnp.bfloat16),
            jax.ShapeDtypeStruct((c_in, c_in), jnp.float32),
            jax.ShapeDtypeStruct((1, c_in), jnp.float32),
        ],
        compiler_params=pltpu.CompilerParams(
            dimension_semantics=(pltpu.ARBITRARY,),
        ),
        cost_estimate=pl.CostEstimate(
            flops=2 * n * p * c_in * c_in,
            transcendentals=0,
            bytes_accessed=n * c_in * (h * w * 4 + p * 2),
        ),
    )(xt)

    m = float(n * p)
    wf = w_mat.astype(jnp.float32)
    mean = (sx @ wf.T) / m
    ey2 = jnp.sum((wf @ gm) * wf, axis=1).reshape(1, c_out) / m
    var = jnp.maximum(ey2 - mean * mean, 0.0)
    inv_std = jax.lax.rsqrt(var + BN_EPS)
    scale = gamma.astype(jnp.float32).reshape(1, c_out) * inv_std
    shift = beta.astype(jnp.float32).reshape(1, c_out) - mean * scale

    tile_n = 4
    while n % tile_n != 0 and tile_n > 1:
        tile_n //= 2
    steps = n // tile_n
    out_flat = pl.pallas_call(
        _conv_bn_kernel,
        grid=(steps,),
        in_specs=[
            pl.BlockSpec((tile_n, p, c_in), lambda i: (i, 0, 0)),
            pl.BlockSpec((c_out, c_in), lambda i: (0, 0)),
            pl.BlockSpec((1, c_out), lambda i: (0, 0)),
            pl.BlockSpec((1, c_out), lambda i: (0, 0)),
        ],
        out_specs=pl.BlockSpec((tile_n, p, c_out), lambda i: (i, 0, 0)),
        out_shape=jax.ShapeDtypeStruct((n, p, c_out), x.dtype),
        compiler_params=pltpu.CompilerParams(
            dimension_semantics=(pltpu.PARALLEL,),
        ),
        cost_estimate=pl.CostEstimate(
            flops=2 * n * p * c_in * c_out + 2 * n * p * c_out,
            transcendentals=0,
            bytes_accessed=n * p * (c_in * 2 + c_out * 4) + c_out * c_in * 2,
        ),
    )(xs, w_mat, scale, shift)

    return out_flat.reshape(n, ho, wo, c_out).transpose(0, 3, 1, 2)


def kernel(x, conv_w, gamma, beta):
    return _forward(x, conv_w, gamma, beta)


# speedup vs baseline: 7.2353x; 2.4565x over previous
import functools

import jax
import jax.numpy as jnp
from jax.experimental import pallas as pl
from jax.experimental.pallas import tpu as pltpu

BN_EPS = 1e-5


def _sub_stats_kernel(xa_ref, xb_ref, xs_ref, g_ref, s_ref, *, ho, wo, w_full):
    i = pl.program_id(0)

    @pl.when(i == 0)
    def _():
        g_ref[...] = jnp.zeros_like(g_ref)
        s_ref[...] = jnp.zeros_like(s_ref)

    c_half = xa_ref.shape[2]
    s_acc = jnp.zeros((1, 2 * c_half), jnp.float32)
    for r in range(ho):
        eva = xa_ref[0, pl.ds(2 * w_full * r, wo, 2), :]
        evb = xb_ref[0, pl.ds(2 * w_full * r, wo, 2), :]
        ev = jnp.concatenate([eva, evb], axis=1)
        xs_ref[0, pl.ds(r * wo, wo), :] = ev.astype(xs_ref.dtype)
        s_acc = s_acc + jnp.sum(ev, axis=0, keepdims=True)
    s_ref[...] += s_acc

    xb = xs_ref[0]
    g_ref[...] += jax.lax.dot_general(
        xb, xb, (((0,), (0,)), ((), ())),
        preferred_element_type=jnp.float32)


def _conv_bn_kernel(xs_ref, w_ref, scale_ref, shift_ref, o_ref):
    tp, nn, c_in = xs_ref.shape
    c_out = w_ref.shape[0]
    xm = xs_ref[...].reshape(tp * nn, c_in)
    y = jax.lax.dot_general(
        xm, w_ref[...], (((1,), (1,)), ((), ())),
        preferred_element_type=jnp.float32)
    y = y * scale_ref[...] + shift_ref[...]
    o_ref[...] = y.reshape(tp, nn, c_out)


@jax.jit
def _forward(x, conv_w, gamma, beta):
    n, c_in, h, w = x.shape
    c_out = conv_w.shape[0]
    ho, wo = h // 2, w // 2
    p = ho * wo

    xt = x.transpose(0, 2, 3, 1).reshape(n, h * w, c_in)
    w_mat = conv_w.reshape(c_out, c_in).astype(jnp.bfloat16)

    xs, gm, sx = pl.pallas_call(
        functools.partial(_sub_stats_kernel, ho=ho, wo=wo, w_full=w),
        grid=(n,),
        in_specs=[
            pl.BlockSpec((1, h * w, 128), lambda i: (i, 0, 0)),
            pl.BlockSpec((1, h * w, 128), lambda i: (i, 0, 1)),
        ],
        out_specs=[
            pl.BlockSpec((1, p, c_in), lambda i: (i, 0, 0)),
            pl.BlockSpec((c_in, c_in), lambda i: (0, 0)),
            pl.BlockSpec((1, c_in), lambda i: (0, 0)),
        ],
        out_shape=[
            jax.ShapeDtypeStruct((n, p, c_in), jnp.bfloat16),
            jax.ShapeDtypeStruct((c_in, c_in), jnp.float32),
            jax.ShapeDtypeStruct((1, c_in), jnp.float32),
        ],
        compiler_params=pltpu.CompilerParams(
            dimension_semantics=(pltpu.ARBITRARY,),
        ),
        cost_estimate=pl.CostEstimate(
            flops=2 * n * p * c_in * c_in,
            transcendentals=0,
            bytes_accessed=n * c_in * (h * w * 4 + p * 2),
        ),
    )(xt, xt)

    m = float(n * p)
    wf = w_mat.astype(jnp.float32)
    mean = (sx @ wf.T) / m
    ey2 = jnp.sum((wf @ gm) * wf, axis=1).reshape(1, c_out) / m
    var = jnp.maximum(ey2 - mean * mean, 0.0)
    inv_std = jax.lax.rsqrt(var + BN_EPS)
    scale = gamma.astype(jnp.float32).reshape(1, c_out) * inv_std
    shift = beta.astype(jnp.float32).reshape(1, c_out) - mean * scale

    xst = xs.transpose(1, 0, 2)
    tile_p = max(d for d in range(1, min(56, p) + 1) if p % d == 0)
    steps = p // tile_p
    out_flat = pl.pallas_call(
        _conv_bn_kernel,
        grid=(steps,),
        in_specs=[
            pl.BlockSpec((tile_p, n, c_in), lambda i: (i, 0, 0)),
            pl.BlockSpec((c_out, c_in), lambda i: (0, 0)),
            pl.BlockSpec((1, c_out), lambda i: (0, 0)),
            pl.BlockSpec((1, c_out), lambda i: (0, 0)),
        ],
        out_specs=pl.BlockSpec((tile_p, n, c_out), lambda i: (i, 0, 0)),
        out_shape=jax.ShapeDtypeStruct((p, n, c_out), x.dtype),
        compiler_params=pltpu.CompilerParams(
            dimension_semantics=(pltpu.PARALLEL,),
        ),
        cost_estimate=pl.CostEstimate(
            flops=2 * n * p * c_in * c_out + 2 * n * p * c_out,
            transcendentals=0,
            bytes_accessed=n * p * (c_in * 2 + c_out * 4) + c_out * c_in * 2,
        ),
    )(xst, w_mat, scale, shift)

    return out_flat.reshape(ho, wo, n, c_out).transpose(2, 3, 0, 1)


def kernel(x, conv_w, gamma, beta):
    return _forward(x, conv_w, gamma, beta)


# speedup vs baseline: 10.6103x; 1.4665x over previous
import functools

import jax
import jax.numpy as jnp
from jax.experimental import pallas as pl
from jax.experimental.pallas import tpu as pltpu

BN_EPS = 1e-5


def _sub_stats_kernel(*refs, wo, n_halves):
    x_refs = refs[:n_halves]
    xst_ref, g_ref, s_ref = refs[n_halves:]
    r = pl.program_id(0)

    @pl.when(r == 0)
    def _():
        g_ref[...] = jnp.zeros_like(g_ref)
        s_ref[...] = jnp.zeros_like(s_ref)

    for j in range(wo):
        for k, href in enumerate(x_refs):
            ev = href[:, 2 * j, :]
            xst_ref[j, :, pl.ds(k * 128, 128)] = ev.astype(xst_ref.dtype)

    nn = xst_ref.shape[1]
    c_in = xst_ref.shape[2]
    xm = xst_ref[...].reshape(wo * nn, c_in)
    g_ref[...] += jax.lax.dot_general(
        xm, xm, (((0,), (0,)), ((), ())),
        preferred_element_type=jnp.float32)
    s_ref[...] += jnp.sum(xm.astype(jnp.float32), axis=0, keepdims=True)


def _conv_bn_kernel(xs_ref, w_ref, scale_ref, shift_ref, o_ref):
    tp, nn, c_in = xs_ref.shape
    c_out = w_ref.shape[0]
    xm = xs_ref[...].reshape(tp * nn, c_in)
    y = jax.lax.dot_general(
        xm, w_ref[...], (((1,), (1,)), ((), ())),
        preferred_element_type=jnp.float32)
    y = y * scale_ref[...] + shift_ref[...]
    o_ref[...] = y.reshape(tp, nn, c_out)


@jax.jit
def _forward(x, conv_w, gamma, beta):
    n, c_in, h, w = x.shape
    c_out = conv_w.shape[0]
    ho, wo = h // 2, w // 2
    p = ho * wo

    xt = x.transpose(0, 2, 3, 1).reshape(n, h * w, c_in)
    w_mat = conv_w.reshape(c_out, c_in).astype(jnp.bfloat16)

    n_halves = c_in // 128

    def _in_idx(k):
        return lambda r: (0, 2 * r, k)

    xst, gm, sx = pl.pallas_call(
        functools.partial(_sub_stats_kernel, wo=wo, n_halves=n_halves),
        grid=(ho,),
        in_specs=[
            pl.BlockSpec((n, w, 128), _in_idx(k)) for k in range(n_halves)
        ],
        out_specs=[
            pl.BlockSpec((wo, n, c_in), lambda r: (r, 0, 0)),
            pl.BlockSpec((c_in, c_in), lambda r: (0, 0)),
            pl.BlockSpec((1, c_in), lambda r: (0, 0)),
        ],
        out_shape=[
            jax.ShapeDtypeStruct((p, n, c_in), jnp.bfloat16),
            jax.ShapeDtypeStruct((c_in, c_in), jnp.float32),
            jax.ShapeDtypeStruct((1, c_in), jnp.float32),
        ],
        compiler_params=pltpu.CompilerParams(
            dimension_semantics=(pltpu.ARBITRARY,),
        ),
        cost_estimate=pl.CostEstimate(
            flops=2 * n * p * c_in * c_in,
            transcendentals=0,
            bytes_accessed=n * c_in * (h * w * 2 + p * 2),
        ),
    )(*([xt] * n_halves))

    m = float(n * p)
    wf = w_mat.astype(jnp.float32)
    mean = (sx @ wf.T) / m
    ey2 = jnp.sum((wf @ gm) * wf, axis=1).reshape(1, c_out) / m
    var = jnp.maximum(ey2 - mean * mean, 0.0)
    inv_std = jax.lax.rsqrt(var + BN_EPS)
    scale = gamma.astype(jnp.float32).reshape(1, c_out) * inv_std
    shift = beta.astype(jnp.float32).reshape(1, c_out) - mean * scale

    tile_p = max(d for d in range(1, min(56, p) + 1) if p % d == 0)
    steps = p // tile_p
    out_flat = pl.pallas_call(
        _conv_bn_kernel,
        grid=(steps,),
        in_specs=[
            pl.BlockSpec((tile_p, n, c_in), lambda i: (i, 0, 0)),
            pl.BlockSpec((c_out, c_in), lambda i: (0, 0)),
            pl.BlockSpec((1, c_out), lambda i: (0, 0)),
            pl.BlockSpec((1, c_out), lambda i: (0, 0)),
        ],
        out_specs=pl.BlockSpec((tile_p, n, c_out), lambda i: (i, 0, 0)),
        out_shape=jax.ShapeDtypeStruct((p, n, c_out), x.dtype),
        compiler_params=pltpu.CompilerParams(
            dimension_semantics=(pltpu.PARALLEL,),
        ),
        cost_estimate=pl.CostEstimate(
            flops=2 * n * p * c_in * c_out + 2 * n * p * c_out,
            transcendentals=0,
            bytes_accessed=n * p * (c_in * 2 + c_out * 4) + c_out * c_in * 2,
        ),
    )(xst, w_mat, scale, shift)

    return out_flat.reshape(ho, wo, n, c_out).transpose(2, 3, 0, 1)


def kernel(x, conv_w, gamma, beta):
    return _forward(x, conv_w, gamma, beta)


# speedup vs baseline: 12.0498x; 1.1357x over previous
import functools

import jax
import jax.numpy as jnp
from jax.experimental import pallas as pl
from jax.experimental.pallas import tpu as pltpu

BN_EPS = 1e-5


def _sub_stats_kernel(*refs, wo, n_halves):
    x_refs = refs[:n_halves]
    xst_ref, g_ref, s_ref = refs[n_halves:]
    r = pl.program_id(0)

    @pl.when(r == 0)
    def _():
        g_ref[...] = jnp.zeros_like(g_ref)
        s_ref[...] = jnp.zeros_like(s_ref)

    rows = []
    for j in range(wo):
        parts = [href[:, 2 * j, :] for href in x_refs]
        rows.append(jnp.concatenate(parts, axis=1))
    blk = jnp.stack(rows, axis=0)
    nn, c_in = blk.shape[1], blk.shape[2]
    xb16 = blk.astype(xst_ref.dtype)
    xst_ref[...] = xb16

    xm = xb16.reshape(wo * nn, c_in)
    g_ref[...] += jax.lax.dot_general(
        xm, xm, (((0,), (0,)), ((), ())),
        preferred_element_type=jnp.float32)
    s_ref[...] += jnp.sum(blk.reshape(wo * nn, c_in), axis=0, keepdims=True)


def _conv_bn_kernel(xs_ref, w_ref, scale_ref, shift_ref, o_ref):
    tp, nn, c_in = xs_ref.shape
    c_out = w_ref.shape[0]
    xm = xs_ref[...].reshape(tp * nn, c_in)
    y = jax.lax.dot_general(
        xm, w_ref[...], (((1,), (1,)), ((), ())),
        preferred_element_type=jnp.float32)
    y = y * scale_ref[...] + shift_ref[...]
    o_ref[...] = y.reshape(tp, nn, c_out)


@jax.jit
def _forward(x, conv_w, gamma, beta):
    n, c_in, h, w = x.shape
    c_out = conv_w.shape[0]
    ho, wo = h // 2, w // 2
    p = ho * wo

    xt = x.transpose(0, 2, 3, 1).reshape(n, h * w, c_in)
    w_mat = conv_w.reshape(c_out, c_in).astype(jnp.bfloat16)

    n_halves = c_in // 128

    def _in_idx(k):
        return lambda r: (0, 2 * r, k)

    xst, gm, sx = pl.pallas_call(
        functools.partial(_sub_stats_kernel, wo=wo, n_halves=n_halves),
        grid=(ho,),
        in_specs=[
            pl.BlockSpec((n, w, 128), _in_idx(k)) for k in range(n_halves)
        ],
        out_specs=[
            pl.BlockSpec((wo, n, c_in), lambda r: (r, 0, 0)),
            pl.BlockSpec((c_in, c_in), lambda r: (0, 0)),
            pl.BlockSpec((1, c_in), lambda r: (0, 0)),
        ],
        out_shape=[
            jax.ShapeDtypeStruct((p, n, c_in), jnp.bfloat16),
            jax.ShapeDtypeStruct((c_in, c_in), jnp.float32),
            jax.ShapeDtypeStruct((1, c_in), jnp.float32),
        ],
        compiler_params=pltpu.CompilerParams(
            dimension_semantics=(pltpu.ARBITRARY,),
        ),
        cost_estimate=pl.CostEstimate(
            flops=2 * n * p * c_in * c_in,
            transcendentals=0,
            bytes_accessed=n * c_in * (h * w * 2 + p * 2),
        ),
    )(*([xt] * n_halves))

    m = float(n * p)
    wf = w_mat.astype(jnp.float32)
    mean = (sx @ wf.T) / m
    ey2 = jnp.sum((wf @ gm) * wf, axis=1).reshape(1, c_out) / m
    var = jnp.maximum(ey2 - mean * mean, 0.0)
    inv_std = jax.lax.rsqrt(var + BN_EPS)
    scale = gamma.astype(jnp.float32).reshape(1, c_out) * inv_std
    shift = beta.astype(jnp.float32).reshape(1, c_out) - mean * scale

    tile_p = max(d for d in range(1, min(56, p) + 1) if p % d == 0)
    steps = p // tile_p
    out_flat = pl.pallas_call(
        _conv_bn_kernel,
        grid=(steps,),
        in_specs=[
            pl.BlockSpec((tile_p, n, c_in), lambda i: (i, 0, 0)),
            pl.BlockSpec((c_out, c_in), lambda i: (0, 0)),
            pl.BlockSpec((1, c_out), lambda i: (0, 0)),
            pl.BlockSpec((1, c_out), lambda i: (0, 0)),
        ],
        out_specs=pl.BlockSpec((tile_p, n, c_out), lambda i: (i, 0, 0)),
        out_shape=jax.ShapeDtypeStruct((p, n, c_out), x.dtype),
        compiler_params=pltpu.CompilerParams(
            dimension_semantics=(pltpu.PARALLEL,),
        ),
        cost_estimate=pl.CostEstimate(
            flops=2 * n * p * c_in * c_out + 2 * n * p * c_out,
            transcendentals=0,
            bytes_accessed=n * p * (c_in * 2 + c_out * 4) + c_out * c_in * 2,
        ),
    )(xst, w_mat, scale, shift)

    return out_flat.reshape(ho, wo, n, c_out).transpose(2, 3, 0, 1)


def kernel(x, conv_w, gamma, beta):
    return _forward(x, conv_w, gamma, beta)
